# baseline (speedup 1.0000x reference)
import functools
import math

import jax
import jax.numpy as jnp
from jax.experimental import pallas as pl
from jax.experimental.pallas import tpu as pltpu

HDIM = 128
CDIM = 128
CORR_LEVELS = 4
CORR_RADIUS = 4
COR_PLANES = CORR_LEVELS * (2 * CORR_RADIUS + 1) ** 2


def _apply_act(r, act):
    if act == "relu":
        return jnp.maximum(r, 0.0)
    if act == "sigmoid":
        return pl.reciprocal(1.0 + jnp.exp(-r), approx=True)
    if act == "tanh":
        return jnp.tanh(r)
    return r


def _tap_conv_kernel(offsets, lout, act, x_ref, w_ref, b_ref, o_ref):
    acc = b_ref[...].astype(jnp.float32)
    for t, off in enumerate(offsets):
        a = x_ref[0, off:off + lout, :]
        acc = acc + jax.lax.dot_general(
            a, w_ref[t], (((1,), (0,)), ((), ())),
            preferred_element_type=jnp.float32)
    o_ref[0] = _apply_act(acc, act).astype(o_ref.dtype)


def conv_tap(x, w3, b, KH, KW, pt, plft, act="none", out_dtype=jnp.bfloat16):
    N, H, W, C = x.shape
    Cout = w3.shape[2]
    pb, prt = KH - 1 - pt, KW - 1 - plft
    Hp, Wp = H + pt + pb, W + plft + prt
    xp = jnp.pad(x.astype(jnp.bfloat16),
                 ((0, 0), (pt, pb), (plft, prt), (0, 0))).reshape(N, Hp * Wp, C)
    lout = H * Wp
    offsets = [ky * Wp + kx for ky in range(KH) for kx in range(KW)]
    lp = ((max(offsets[-1] + lout, Hp * Wp) + 7) // 8) * 8
    if lp > Hp * Wp:
        xp = jnp.pad(xp, ((0, 0), (0, lp - Hp * Wp), (0, 0)))
    out = pl.pallas_call(
        functools.partial(_tap_conv_kernel, offsets, lout, act),
        out_shape=jax.ShapeDtypeStruct((N, lout, Cout), out_dtype),
        grid=(N,),
        in_specs=[pl.BlockSpec((1, lp, C), lambda i: (i, 0, 0)),
                  pl.BlockSpec((KH * KW, C, Cout), lambda i: (0, 0, 0)),
                  pl.BlockSpec((1, Cout), lambda i: (0, 0))],
        out_specs=pl.BlockSpec((1, lout, Cout), lambda i: (i, 0, 0)),
        compiler_params=pltpu.CompilerParams(
            dimension_semantics=("parallel",)),
    )(xp, w3.astype(jnp.bfloat16), b)
    out = out.reshape(N, H, Wp, Cout)
    if Wp != W:
        out = out[:, :, :W, :]
    return out


def _unflatten_w(w, KH, KW, C):
    return w[:KH * KW * C].reshape(KH * KW, C, w.shape[1])


def _mm_kernel(act, a_ref, w_ref, b_ref, o_ref):
    r = jnp.dot(a_ref[...], w_ref[...], preferred_element_type=jnp.float32)
    o_ref[...] = _apply_act(r + b_ref[...], act).astype(o_ref.dtype)


def _pick_row_tile(m):
    for t in (512, 256, 128):
        if m % t == 0:
            return t, m
    if m % 8 == 0 and m <= 1024:
        return m, m
    return 128, ((m + 127) // 128) * 128


def matmul_bias_act(a, w, b, act="none", out_dtype=jnp.bfloat16):
    M, K = a.shape
    Nn = w.shape[1]
    TM, Mp = _pick_row_tile(M)
    if Mp != M:
        a = jnp.pad(a, ((0, Mp - M), (0, 0)))
    a = a.astype(jnp.bfloat16)
    TN = Nn if Nn % 128 else Nn
    if Nn % 128 == 0:
        TN = 256 if Nn % 256 == 0 else 128
    out = pl.pallas_call(
        functools.partial(_mm_kernel, act),
        out_shape=jax.ShapeDtypeStruct((Mp, Nn), out_dtype),
        grid=(Mp // TM, Nn // TN),
        in_specs=[pl.BlockSpec((TM, K), lambda i, j: (i, 0)),
                  pl.BlockSpec((K, TN), lambda i, j: (0, j)),
                  pl.BlockSpec((1, TN), lambda i, j: (0, j))],
        out_specs=pl.BlockSpec((TM, TN), lambda i, j: (i, j)),
        compiler_params=pltpu.CompilerParams(
            dimension_semantics=("parallel", "parallel")),
    )(a, w, b)
    return out[:M] if Mp != M else out


def conv2d_im2col(x, wp, ksize, stride=(1, 1), padding=(0, 0), act="none",
                  out_dtype=jnp.bfloat16):
    w, b = wp["w"], wp["b"]
    KH, KW = ksize
    N, H, W, Cin = x.shape
    sh, sw = stride
    ph, pw = padding
    Ho = (H + 2 * ph - KH) // sh + 1
    Wo = (W + 2 * pw - KW) // sw + 1
    M = N * Ho * Wo
    Kp = w.shape[0]
    if KH == 1 and KW == 1 and stride == (1, 1):
        a = x.reshape(M, Cin)
        if Kp != Cin:
            a = jnp.pad(a, ((0, 0), (0, Kp - Cin)))
    else:
        xp = jnp.pad(x, ((0, 0), (ph, ph), (pw, pw), (0, 0)))
        cols = [xp[:, ky:ky + sh * (Ho - 1) + 1:sh,
                   kx:kx + sw * (Wo - 1) + 1:sw, :]
                for ky in range(KH) for kx in range(KW)]
        kpad = Kp - KH * KW * Cin
        if kpad:
            cols.append(jnp.zeros((N, Ho, Wo, kpad), x.dtype))
        a = jnp.concatenate(cols, axis=-1).reshape(M, Kp)
    out = matmul_bias_act(a, w, b, act=act, out_dtype=out_dtype)
    return out.reshape(N, Ho, Wo, w.shape[1])


def conv2d(x, wp, ksize, stride=(1, 1), padding=(0, 0), act="none",
           out_dtype=jnp.bfloat16):
    KH, KW = ksize
    Cin = x.shape[3]
    if stride == (1, 1) and (KH, KW) != (1, 1) and Cin >= 64:
        w3 = _unflatten_w(wp["w"], KH, KW, Cin)
        return conv_tap(x, w3, wp["b"], KH, KW, padding[0], padding[1],
                        act=act, out_dtype=out_dtype)
    return conv2d_im2col(x, wp, ksize, stride, padding, act, out_dtype)


def _ctx_act_kernel(c_ref, net_ref, inp_ref):
    c = c_ref[...].astype(jnp.float32)
    net_ref[...] = jnp.tanh(c[:, :HDIM]).astype(net_ref.dtype)
    inp_ref[...] = jnp.maximum(c[:, HDIM:], 0.0).astype(inp_ref.dtype)


def _gru_rh_kernel(zr_ref, h_ref, rh_ref):
    r = zr_ref[:, HDIM:].astype(jnp.float32)
    rh_ref[...] = (r * h_ref[...].astype(jnp.float32)).astype(rh_ref.dtype)


def _gru_blend_kernel(zr_ref, q_ref, h_ref, ho_ref):
    z = zr_ref[:, :HDIM].astype(jnp.float32)
    q = q_ref[...].astype(jnp.float32)
    h = h_ref[...].astype(jnp.float32)
    ho_ref[...] = ((1.0 - z) * h + z * q).astype(ho_ref.dtype)


def _row_call(row_kernel, ins, out_widths, out_dtypes):
    M = ins[0].shape[0]
    TR, Mp = _pick_row_tile(M)
    if Mp != M:
        ins = [jnp.pad(x, ((0, Mp - M), (0, 0))) for x in ins]
    outs = pl.pallas_call(
        row_kernel,
        out_shape=tuple(jax.ShapeDtypeStruct((Mp, w), d)
                        for w, d in zip(out_widths, out_dtypes)),
        grid=(Mp // TR,),
        in_specs=[pl.BlockSpec((TR, x.shape[1]), lambda i: (i, 0)) for x in ins],
        out_specs=tuple(pl.BlockSpec((TR, w), lambda i: (i, 0))
                        for w in out_widths),
        compiler_params=pltpu.CompilerParams(dimension_semantics=("parallel",)),
    )(*ins)
    if not isinstance(outs, (tuple, list)):
        outs = (outs,)
    if Mp != M:
        outs = tuple(o[:M] for o in outs)
    return tuple(outs)


def norm_relu(x, mode):
    x = x.astype(jnp.float32)
    if mode == "instance":
        mean = x.mean(axis=(1, 2), keepdims=True)
        var = x.var(axis=(1, 2), keepdims=True)
        x = (x - mean) * jax.lax.rsqrt(var + 1e-5)
    elif mode == "batch":
        mean = x.mean(axis=(0, 1, 2), keepdims=True)
        var = x.var(axis=(0, 1, 2), keepdims=True)
        x = (x - mean) * jax.lax.rsqrt(var + 1e-5)
    return jnp.maximum(x, 0.0).astype(jnp.bfloat16)


def encoder_forward(p, x, norm):
    x = norm_relu(conv2d(x, p["c1"], (7, 7), stride=(2, 2), padding=(3, 3)),
                  norm)
    x = norm_relu(conv2d(x, p["c2"], (3, 3), stride=(2, 2), padding=(1, 1)),
                  norm)
    x = norm_relu(conv2d(x, p["c3"], (3, 3), stride=(2, 2), padding=(1, 1)),
                  norm)
    return conv2d(x, p["c4"], (1, 1))


def coords_grid(N, H, W):
    ys, xs = jnp.meshgrid(jnp.arange(H, dtype=jnp.float32),
                          jnp.arange(W, dtype=jnp.float32), indexing="ij")
    coords = jnp.stack([xs, ys], axis=0)
    return jnp.broadcast_to(coords[None], (N, 2, H, W))


def build_f2_pyramid(fmap2, num_levels=CORR_LEVELS):
    N, H, W, C = fmap2.shape
    levels = [fmap2.astype(jnp.bfloat16).reshape(N, H * W, C)]
    cur = fmap2.astype(jnp.float32)
    for _ in range(num_levels - 1):
        _, h, w, _ = cur.shape
        cur = cur.reshape(N, h // 2, 2, w // 2, 2, C).mean(axis=(2, 4))
        levels.append(cur.astype(jnp.bfloat16).reshape(N, (h // 2) * (w // 2),
                                                       C))
    return levels


def _corr_lookup_kernel(radius, shapes, scale, c_ref, f1_ref,
                        g0_ref, g1_ref, g2_ref, g3_ref, o_ref):
    R = 2 * radius + 1
    f1 = f1_ref[0]
    cx = c_ref[0, 0:1, :]
    cy = c_ref[0, 1:2, :]
    rows = []
    for lvl, g_ref in enumerate((g0_ref, g1_ref, g2_ref, g3_ref)):
        Hl, Wl = shapes[lvl]
        m2d = jax.lax.dot_general(
            g_ref[0], f1, (((1,), (1,)), ((), ())),
            preferred_element_type=jnp.float32) * scale
        inv = 1.0 / (2.0 ** lvl)
        cxl = cx * inv
        cyl = cy * inv
        fx = jnp.floor(cxl)
        fy = jnp.floor(cyl)
        wx1 = cxl - fx
        wx0 = 1.0 - wx1
        wy1 = cyl - fy
        wy0 = 1.0 - wy1
        fxi = fx.astype(jnp.int32)
        fyi = fy.astype(jnp.int32)
        if Wl >= 8:
            m = m2d.reshape(Hl, Wl, m2d.shape[1])
            ys = jax.lax.broadcasted_iota(jnp.int32, (Hl, 1, 1), 0)
            xs = jax.lax.broadcasted_iota(jnp.int32, (1, Wl, 1), 1)
            t1 = []
            for b in range(R):
                ty = (fyi + (b - radius))[None]
                sel = (wy0[None] * (ys == ty) + wy1[None] * (ys == ty + 1))
                t1.append(jnp.sum(m * sel, axis=0))
            t1 = jnp.stack(t1, axis=0)
            for a in range(R):
                tx = (fxi + (a - radius))[None]
                sel = (wx0[None] * (xs == tx) + wx1[None] * (xs == tx + 1))
                rows.append(jnp.sum(t1 * sel, axis=1))
        else:
            kk = jax.lax.broadcasted_iota(jnp.int32, (Hl * Wl, 1), 0)
            ky = kk // Wl
            kx = kk % Wl
            for a in range(R):
                tx = fxi + (a - radius)
                sx = wx0 * (kx == tx) + wx1 * (kx == tx + 1)
                vals = []
                for b in range(R):
                    ty = fyi + (b - radius)
                    sy = wy0 * (ky == ty) + wy1 * (ky == ty + 1)
                    vals.append(jnp.sum(m2d * sx * sy, axis=0))
                rows.append(jnp.stack(vals, axis=0))
    o_ref[0] = jnp.concatenate(rows, axis=0).astype(o_ref.dtype)


def corr_lookup(f1, f2_levels, coords, radius=CORR_RADIUS):
    N, _, H, W = coords.shape
    HW = H * W
    C = f1.shape[2]
    R = 2 * radius + 1
    nplanes = len(f2_levels) * R * R
    shapes = []
    h, w = H, W
    for _ in f2_levels:
        shapes.append((h, w))
        h, w = h // 2, w // 2
    shapes = tuple(shapes)
    coords_q = coords.reshape(N, 2, HW)
    scale = 1.0 / math.sqrt(C)
    g_specs = [pl.BlockSpec((1, g.shape[1], C), lambda n: (n, 0, 0))
               for g in f2_levels]
    out = pl.pallas_call(
        functools.partial(_corr_lookup_kernel, radius, shapes, scale),
        out_shape=jax.ShapeDtypeStruct((N, nplanes, HW), jnp.bfloat16),
        grid=(N,),
        in_specs=[pl.BlockSpec((1, 2, HW), lambda n: (n, 0, 0)),
                  pl.BlockSpec((1, HW, C), lambda n: (n, 0, 0))] + g_specs,
        out_specs=pl.BlockSpec((1, nplanes, HW), lambda n: (n, 0, 0)),
        compiler_params=pltpu.CompilerParams(
            dimension_semantics=("parallel",)),
    )(coords_q, f1, *f2_levels)
    return jnp.transpose(out, (0, 2, 1)).reshape(N, H, W, nplanes)


def ctx_act(c2d):
    return _row_call(_ctx_act_kernel, [c2d], (HDIM, CDIM),
                     (jnp.bfloat16, jnp.bfloat16))


def sep_conv_gru_dir(prep, h, x, idx, ksize, padding):
    N, H, W, _ = h.shape
    M = N * H * W
    hx = jnp.concatenate([h, x], axis=-1)
    zr = conv2d(hx, prep["zr" + idx], ksize, padding=padding, act="sigmoid")
    zr2 = zr.reshape(M, 2 * HDIM)
    h2 = h.reshape(M, HDIM)
    (rh,) = _row_call(_gru_rh_kernel, [zr2, h2], (HDIM,), (jnp.bfloat16,))
    q_in = jnp.concatenate([rh.reshape(N, H, W, HDIM), x], axis=-1)
    qt = conv2d(q_in, prep["q" + idx], ksize, padding=padding, act="tanh")
    (hn,) = _row_call(_gru_blend_kernel, [zr2, qt.reshape(M, HDIM), h2],
                      (HDIM,), (jnp.bfloat16,))
    return hn.reshape(N, H, W, HDIM)


def update_block(prep, net, inp, corr, flow, info):
    fi = jnp.concatenate([jnp.transpose(flow, (0, 2, 3, 1)),
                          jnp.transpose(info, (0, 2, 3, 1))], axis=-1)
    fi_b = fi.astype(jnp.bfloat16)
    corr_b = corr.astype(jnp.bfloat16)
    cor = conv2d(corr_b, prep["convc1"], (1, 1), act="relu")
    cor = conv2d(cor, prep["convc2"], (3, 3), padding=(1, 1), act="relu")
    flo = conv2d(fi_b, prep["convf1"], (7, 7), padding=(3, 3), act="relu")
    flo = conv2d(flo, prep["convf2"], (3, 3), padding=(1, 1), act="relu")
    mot = conv2d(jnp.concatenate([cor, flo], -1), prep["conv"], (3, 3),
                 padding=(1, 1), act="relu")
    x = jnp.concatenate([inp, mot, fi_b], axis=-1)
    net = sep_conv_gru_dir(prep, net, x, "1", (1, 5), (0, 2))
    net = sep_conv_gru_dir(prep, net, x, "2", (5, 1), (2, 0))
    dm = conv2d(net, prep["dm"], (3, 3), padding=(1, 1), act="relu")
    delta = conv2d(dm[..., :256], prep["fh2"], (3, 3), padding=(1, 1),
                   act="none", out_dtype=jnp.float32)
    mask = conv2d(dm[..., 256:], prep["mh2"], (1, 1), act="none",
                  out_dtype=jnp.float32)
    return (net, jnp.transpose(mask, (0, 3, 1, 2)),
            jnp.transpose(delta, (0, 3, 1, 2)))


def unfold3x3(x):
    N, C, H, W = x.shape
    xp = jnp.pad(x, ((0, 0), (0, 0), (1, 1), (1, 1)))
    cols = [xp[:, :, ky:ky + H, kx:kx + W] for ky in range(3) for kx in range(3)]
    return jnp.stack(cols, axis=2)


def _upsample_kernel(m_ref, uf_ref, ui_ref, of_ref, oi_ref):
    m = m_ref[...]
    m = m - jnp.max(m, axis=0, keepdims=True)
    e = jnp.exp(m)
    sm = e * pl.reciprocal(jnp.sum(e, axis=0, keepdims=True), approx=True)
    uf = uf_ref[...]
    ui = ui_ref[...]
    for c in range(2):
        of_ref[c, :, :] = jnp.sum(sm * uf[:, c, :][:, None, :], axis=0)
        oi_ref[c, :, :] = jnp.sum(sm * ui[:, c, :][:, None, :], axis=0)


def upsample_flow(flow, info, mask):
    N, _, H, W = flow.shape
    P = N * H * W
    mask_k = jnp.transpose(mask.reshape(N, 9, 64, H, W),
                           (1, 2, 0, 3, 4)).reshape(9, 64, P)
    uf = unfold3x3(8.0 * flow)
    ui = unfold3x3(info)
    uf_k = jnp.transpose(uf, (2, 1, 0, 3, 4)).reshape(9, 2, P)
    ui_k = jnp.transpose(ui, (2, 1, 0, 3, 4)).reshape(9, 2, P)
    TP = 256 if P % 256 == 0 else 128
    of, oi = pl.pallas_call(
        _upsample_kernel,
        out_shape=(jax.ShapeDtypeStruct((2, 64, P), jnp.float32),
                   jax.ShapeDtypeStruct((2, 64, P), jnp.float32)),
        grid=(P // TP,),
        in_specs=[pl.BlockSpec((9, 64, TP), lambda i: (0, 0, i)),
                  pl.BlockSpec((9, 2, TP), lambda i: (0, 0, i)),
                  pl.BlockSpec((9, 2, TP), lambda i: (0, 0, i))],
        out_specs=(pl.BlockSpec((2, 64, TP), lambda i: (0, 0, i)),
                   pl.BlockSpec((2, 64, TP), lambda i: (0, 0, i))),
        compiler_params=pltpu.CompilerParams(dimension_semantics=("parallel",)),
    )(mask_k, uf_k, ui_k)

    def finish(o):
        o = o.reshape(2, 8, 8, N, H, W)
        o = jnp.transpose(o, (3, 0, 4, 1, 5, 2))
        return o.reshape(N, 2, 8 * H, 8 * W)

    return finish(of), finish(oi)


def raft_forward(prep, image1, image2, iters=2):
    N = image1.shape[0]
    x = jnp.transpose(jnp.concatenate([image1, image2], axis=0),
                      (0, 2, 3, 1)).astype(jnp.bfloat16)
    fmaps = encoder_forward(prep["fnet"], x, "instance")
    fmap1, fmap2 = fmaps[:N], fmaps[N:]
    cnet = encoder_forward(prep["cnet"],
                           jnp.transpose(image1, (0, 2, 3, 1)).astype(jnp.bfloat16),
                           "batch")
    H8, W8 = cnet.shape[1], cnet.shape[2]
    net2d, inp2d = ctx_act(cnet.reshape(N * H8 * W8, HDIM + CDIM))
    net = net2d.reshape(N, H8, W8, HDIM)
    inp = inp2d.reshape(N, H8, W8, CDIM)

    f1 = fmap1.astype(jnp.bfloat16).reshape(N, H8 * W8, -1)
    f2_levels = build_f2_pyramid(fmap2)
    coords0 = coords_grid(N, H8, W8)
    coords1 = coords0
    info = jnp.zeros_like(coords1)

    flow_predictions, info_predictions = [], []
    for _ in range(iters):
        corr = corr_lookup(f1, f2_levels, coords1, radius=CORR_RADIUS)
        flow = coords1 - coords0
        net, up_mask, delta = update_block(prep, net, inp, corr, flow, info)
        coords1 = coords1 + delta[:, :2]
        info = info + delta[:, 2:]
        flow_up, info_up = upsample_flow(coords1 - coords0, info, up_mask)
        flow_predictions.append(flow_up)
        info_predictions.append(info_up)
    return flow_predictions, info_predictions


def kernel(image1, image2,
           fnet_c1_w, fnet_c1_b, fnet_c2_w, fnet_c2_b,
           fnet_c3_w, fnet_c3_b, fnet_c4_w, fnet_c4_b,
           cnet_c1_w, cnet_c1_b, cnet_c2_w, cnet_c2_b,
           cnet_c3_w, cnet_c3_b, cnet_c4_w, cnet_c4_b,
           convc1_w, convc1_b, convc2_w, convc2_b,
           convf1_w, convf1_b, convf2_w, convf2_b,
           conv_w, conv_b, fh2_w, fh2_b,
           zr1_w, zr1_b, zr2_w, zr2_b, q1_w, q1_b, q2_w, q2_b,
           dm_w, dm_b, mh2_w, mh2_b):
    prep = {
        "fnet": {"c1": {"w": fnet_c1_w, "b": fnet_c1_b},
                 "c2": {"w": fnet_c2_w, "b": fnet_c2_b},
                 "c3": {"w": fnet_c3_w, "b": fnet_c3_b},
                 "c4": {"w": fnet_c4_w, "b": fnet_c4_b}},
        "cnet": {"c1": {"w": cnet_c1_w, "b": cnet_c1_b},
                 "c2": {"w": cnet_c2_w, "b": cnet_c2_b},
                 "c3": {"w": cnet_c3_w, "b": cnet_c3_b},
                 "c4": {"w": cnet_c4_w, "b": cnet_c4_b}},
        "convc1": {"w": convc1_w, "b": convc1_b},
        "convc2": {"w": convc2_w, "b": convc2_b},
        "convf1": {"w": convf1_w, "b": convf1_b},
        "convf2": {"w": convf2_w, "b": convf2_b},
        "conv": {"w": conv_w, "b": conv_b},
        "fh2": {"w": fh2_w, "b": fh2_b},
        "zr1": {"w": zr1_w, "b": zr1_b},
        "zr2": {"w": zr2_w, "b": zr2_b},
        "q1": {"w": q1_w, "b": q1_b},
        "q2": {"w": q2_w, "b": q2_b},
        "dm": {"w": dm_w, "b": dm_b},
        "mh2": {"w": mh2_w, "b": mh2_b},
    }
    return raft_forward(prep, image1, image2, iters=2)


# speedup vs baseline: 1.8067x; 1.8067x over previous
import functools
import math

import jax
import jax.numpy as jnp
from jax.experimental import pallas as pl
from jax.experimental.pallas import tpu as pltpu

HDIM = 128
CDIM = 128
CORR_LEVELS = 4
CORR_RADIUS = 4
COR_PLANES = CORR_LEVELS * (2 * CORR_RADIUS + 1) ** 2


def _apply_act(r, act):
    if act == "relu":
        return jnp.maximum(r, 0.0)
    if act == "sigmoid":
        return pl.reciprocal(1.0 + jnp.exp(-r), approx=True)
    if act == "tanh":
        return jnp.tanh(r)
    return r


def _tap_conv_kernel(offsets, lout, act, x_ref, w_ref, b_ref, o_ref):
    acc = b_ref[...].astype(jnp.float32)
    for t, off in enumerate(offsets):
        a = x_ref[0, off:off + lout, :]
        acc = acc + jax.lax.dot_general(
            a, w_ref[t], (((1,), (0,)), ((), ())),
            preferred_element_type=jnp.float32)
    o_ref[0] = _apply_act(acc, act).astype(o_ref.dtype)


def conv_tap(x, w3, b, KH, KW, pt, plft, act="none", out_dtype=jnp.bfloat16):
    N, H, W, C = x.shape
    Cout = w3.shape[2]
    pb, prt = KH - 1 - pt, KW - 1 - plft
    Hp, Wp = H + pt + pb, W + plft + prt
    xp = jnp.pad(x.astype(jnp.bfloat16),
                 ((0, 0), (pt, pb), (plft, prt), (0, 0))).reshape(N, Hp * Wp, C)
    lout = H * Wp
    offsets = [ky * Wp + kx for ky in range(KH) for kx in range(KW)]
    lp = ((max(offsets[-1] + lout, Hp * Wp) + 7) // 8) * 8
    if lp > Hp * Wp:
        xp = jnp.pad(xp, ((0, 0), (0, lp - Hp * Wp), (0, 0)))
    out = pl.pallas_call(
        functools.partial(_tap_conv_kernel, offsets, lout, act),
        out_shape=jax.ShapeDtypeStruct((N, lout, Cout), out_dtype),
        grid=(N,),
        in_specs=[pl.BlockSpec((1, lp, C), lambda i: (i, 0, 0)),
                  pl.BlockSpec((KH * KW, C, Cout), lambda i: (0, 0, 0)),
                  pl.BlockSpec((1, Cout), lambda i: (0, 0))],
        out_specs=pl.BlockSpec((1, lout, Cout), lambda i: (i, 0, 0)),
        compiler_params=pltpu.CompilerParams(
            dimension_semantics=("parallel",)),
    )(xp, w3.astype(jnp.bfloat16), b)
    out = out.reshape(N, H, Wp, Cout)
    if Wp != W:
        out = out[:, :, :W, :]
    return out


def _unflatten_w(w, KH, KW, C):
    return w[:KH * KW * C].reshape(KH * KW, C, w.shape[1])


def _mm_kernel(act, a_ref, w_ref, b_ref, o_ref):
    r = jnp.dot(a_ref[...], w_ref[...], preferred_element_type=jnp.float32)
    o_ref[...] = _apply_act(r + b_ref[...], act).astype(o_ref.dtype)


def _pick_row_tile(m):
    for t in (512, 256, 128):
        if m % t == 0:
            return t, m
    if m % 8 == 0 and m <= 1024:
        return m, m
    return 128, ((m + 127) // 128) * 128


def matmul_bias_act(a, w, b, act="none", out_dtype=jnp.bfloat16):
    M, K = a.shape
    Nn = w.shape[1]
    TM, Mp = _pick_row_tile(M)
    if Mp != M:
        a = jnp.pad(a, ((0, Mp - M), (0, 0)))
    a = a.astype(jnp.bfloat16)
    TN = Nn if Nn % 128 else Nn
    if Nn % 128 == 0:
        TN = 256 if Nn % 256 == 0 else 128
    out = pl.pallas_call(
        functools.partial(_mm_kernel, act),
        out_shape=jax.ShapeDtypeStruct((Mp, Nn), out_dtype),
        grid=(Mp // TM, Nn // TN),
        in_specs=[pl.BlockSpec((TM, K), lambda i, j: (i, 0)),
                  pl.BlockSpec((K, TN), lambda i, j: (0, j)),
                  pl.BlockSpec((1, TN), lambda i, j: (0, j))],
        out_specs=pl.BlockSpec((TM, TN), lambda i, j: (i, j)),
        compiler_params=pltpu.CompilerParams(
            dimension_semantics=("parallel", "parallel")),
    )(a, w, b)
    return out[:M] if Mp != M else out


def conv2d_im2col(x, wp, ksize, stride=(1, 1), padding=(0, 0), act="none",
                  out_dtype=jnp.bfloat16):
    w, b = wp["w"], wp["b"]
    KH, KW = ksize
    N, H, W, Cin = x.shape
    sh, sw = stride
    ph, pw = padding
    Ho = (H + 2 * ph - KH) // sh + 1
    Wo = (W + 2 * pw - KW) // sw + 1
    M = N * Ho * Wo
    Kp = w.shape[0]
    if KH == 1 and KW == 1 and stride == (1, 1):
        a = x.reshape(M, Cin)
        if Kp != Cin:
            a = jnp.pad(a, ((0, 0), (0, Kp - Cin)))
    else:
        xp = jnp.pad(x, ((0, 0), (ph, ph), (pw, pw), (0, 0)))
        cols = [xp[:, ky:ky + sh * (Ho - 1) + 1:sh,
                   kx:kx + sw * (Wo - 1) + 1:sw, :]
                for ky in range(KH) for kx in range(KW)]
        kpad = Kp - KH * KW * Cin
        if kpad:
            cols.append(jnp.zeros((N, Ho, Wo, kpad), x.dtype))
        a = jnp.concatenate(cols, axis=-1).reshape(M, Kp)
    out = matmul_bias_act(a, w, b, act=act, out_dtype=out_dtype)
    return out.reshape(N, Ho, Wo, w.shape[1])


def conv2d_s2(x, wp, k, act="none", out_dtype=jnp.bfloat16):
    w, b = wp["w"], wp["b"]
    N, H, W, C = x.shape
    Cout = w.shape[1]
    p = (k - 1) // 2
    w3 = w[:k * k * C].reshape(k, k, C, Cout)
    NT = p + 1
    pt2 = (p + 1) // 2
    wt = jnp.zeros((NT, NT, 2, 2, C, Cout), w.dtype)
    for j in range(k):
        ty, py = (j - p) // 2 + pt2, (j - p) % 2
        for i in range(k):
            tx, px = (i - p) // 2 + pt2, (i - p) % 2
            wt = wt.at[ty, tx, py, px].set(w3[j, i])
    wt = wt.reshape(NT * NT, 4 * C, Cout)
    H2, W2 = H // 2, W // 2
    xs = x.reshape(N, H2, 2, W2, 2, C).transpose(0, 1, 3, 2, 4, 5)
    xs = xs.reshape(N, H2, W2, 4 * C)
    return conv_tap(xs, wt, b, NT, NT, pt2, pt2, act=act, out_dtype=out_dtype)


def conv2d(x, wp, ksize, stride=(1, 1), padding=(0, 0), act="none",
           out_dtype=jnp.bfloat16):
    KH, KW = ksize
    N, H, W, Cin = x.shape
    if stride == (1, 1) and (KH, KW) != (1, 1) and Cin >= 64:
        w3 = _unflatten_w(wp["w"], KH, KW, Cin)
        return conv_tap(x, w3, wp["b"], KH, KW, padding[0], padding[1],
                        act=act, out_dtype=out_dtype)
    if (stride == (2, 2) and KH == KW and KH % 2 == 1 and Cin >= 16
            and padding == ((KH - 1) // 2, (KW - 1) // 2)
            and H % 2 == 0 and W % 2 == 0):
        return conv2d_s2(x, wp, KH, act=act, out_dtype=out_dtype)
    return conv2d_im2col(x, wp, ksize, stride, padding, act, out_dtype)


def _ctx_act_kernel(c_ref, net_ref, inp_ref):
    c = c_ref[...].astype(jnp.float32)
    net_ref[...] = jnp.tanh(c[:, :HDIM]).astype(net_ref.dtype)
    inp_ref[...] = jnp.maximum(c[:, HDIM:], 0.0).astype(inp_ref.dtype)


def _gru_rh_kernel(zr_ref, h_ref, rh_ref):
    r = zr_ref[:, HDIM:].astype(jnp.float32)
    rh_ref[...] = (r * h_ref[...].astype(jnp.float32)).astype(rh_ref.dtype)


def _gru_blend_kernel(zr_ref, q_ref, h_ref, ho_ref):
    z = zr_ref[:, :HDIM].astype(jnp.float32)
    q = q_ref[...].astype(jnp.float32)
    h = h_ref[...].astype(jnp.float32)
    ho_ref[...] = ((1.0 - z) * h + z * q).astype(ho_ref.dtype)


def _row_call(row_kernel, ins, out_widths, out_dtypes):
    M = ins[0].shape[0]
    TR, Mp = _pick_row_tile(M)
    if Mp != M:
        ins = [jnp.pad(x, ((0, Mp - M), (0, 0))) for x in ins]
    outs = pl.pallas_call(
        row_kernel,
        out_shape=tuple(jax.ShapeDtypeStruct((Mp, w), d)
                        for w, d in zip(out_widths, out_dtypes)),
        grid=(Mp // TR,),
        in_specs=[pl.BlockSpec((TR, x.shape[1]), lambda i: (i, 0)) for x in ins],
        out_specs=tuple(pl.BlockSpec((TR, w), lambda i: (i, 0))
                        for w in out_widths),
        compiler_params=pltpu.CompilerParams(dimension_semantics=("parallel",)),
    )(*ins)
    if not isinstance(outs, (tuple, list)):
        outs = (outs,)
    if Mp != M:
        outs = tuple(o[:M] for o in outs)
    return tuple(outs)


def norm_relu(x, mode):
    x = x.astype(jnp.float32)
    if mode == "instance":
        mean = x.mean(axis=(1, 2), keepdims=True)
        var = x.var(axis=(1, 2), keepdims=True)
        x = (x - mean) * jax.lax.rsqrt(var + 1e-5)
    elif mode == "batch":
        mean = x.mean(axis=(0, 1, 2), keepdims=True)
        var = x.var(axis=(0, 1, 2), keepdims=True)
        x = (x - mean) * jax.lax.rsqrt(var + 1e-5)
    return jnp.maximum(x, 0.0).astype(jnp.bfloat16)


def encoder_forward(p, x, norm):
    x = norm_relu(conv2d(x, p["c1"], (7, 7), stride=(2, 2), padding=(3, 3)),
                  norm)
    x = norm_relu(conv2d(x, p["c2"], (3, 3), stride=(2, 2), padding=(1, 1)),
                  norm)
    x = norm_relu(conv2d(x, p["c3"], (3, 3), stride=(2, 2), padding=(1, 1)),
                  norm)
    return conv2d(x, p["c4"], (1, 1))


def coords_grid(N, H, W):
    ys, xs = jnp.meshgrid(jnp.arange(H, dtype=jnp.float32),
                          jnp.arange(W, dtype=jnp.float32), indexing="ij")
    coords = jnp.stack([xs, ys], axis=0)
    return jnp.broadcast_to(coords[None], (N, 2, H, W))


def build_f2_pyramid(fmap2, num_levels=CORR_LEVELS):
    N, H, W, C = fmap2.shape
    levels = [fmap2.astype(jnp.bfloat16).reshape(N, H * W, C)]
    cur = fmap2.astype(jnp.float32)
    for _ in range(num_levels - 1):
        _, h, w, _ = cur.shape
        cur = cur.reshape(N, h // 2, 2, w // 2, 2, C).mean(axis=(2, 4))
        levels.append(cur.astype(jnp.bfloat16).reshape(N, (h // 2) * (w // 2),
                                                       C))
    return levels


def _corr_lookup_kernel(radius, shapes, scale, c_ref, f1_ref,
                        g0_ref, g1_ref, g2_ref, g3_ref, o_ref):
    R = 2 * radius + 1
    f1 = f1_ref[0]
    cx = c_ref[0, 0:1, :]
    cy = c_ref[0, 1:2, :]
    rows = []
    for lvl, g_ref in enumerate((g0_ref, g1_ref, g2_ref, g3_ref)):
        Hl, Wl = shapes[lvl]
        m2d = jax.lax.dot_general(
            g_ref[0], f1, (((1,), (1,)), ((), ())),
            preferred_element_type=jnp.float32) * scale
        inv = 1.0 / (2.0 ** lvl)
        cxl = cx * inv
        cyl = cy * inv
        fx = jnp.floor(cxl)
        fy = jnp.floor(cyl)
        wx1 = cxl - fx
        wx0 = 1.0 - wx1
        wy1 = cyl - fy
        wy0 = 1.0 - wy1
        fxi = fx.astype(jnp.int32)
        fyi = fy.astype(jnp.int32)
        if Wl >= 8:
            m = m2d.reshape(Hl, Wl, m2d.shape[1])
            ys = jax.lax.broadcasted_iota(jnp.int32, (Hl, 1, 1), 0)
            xs = jax.lax.broadcasted_iota(jnp.int32, (1, Wl, 1), 1)
            t1 = []
            for b in range(R):
                ty = (fyi + (b - radius))[None]
                sel = (wy0[None] * (ys == ty) + wy1[None] * (ys == ty + 1))
                t1.append(jnp.sum(m * sel, axis=0))
            t1 = jnp.stack(t1, axis=0)
            for a in range(R):
                tx = (fxi + (a - radius))[None]
                sel = (wx0[None] * (xs == tx) + wx1[None] * (xs == tx + 1))
                rows.append(jnp.sum(t1 * sel, axis=1))
        else:
            kk = jax.lax.broadcasted_iota(jnp.int32, (Hl * Wl, 1), 0)
            ky = kk // Wl
            kx = kk % Wl
            for a in range(R):
                tx = fxi + (a - radius)
                sx = wx0 * (kx == tx) + wx1 * (kx == tx + 1)
                vals = []
                for b in range(R):
                    ty = fyi + (b - radius)
                    sy = wy0 * (ky == ty) + wy1 * (ky == ty + 1)
                    vals.append(jnp.sum(m2d * sx * sy, axis=0))
                rows.append(jnp.stack(vals, axis=0))
    o_ref[0] = jnp.concatenate(rows, axis=0).astype(o_ref.dtype)


def corr_lookup(f1, f2_levels, coords, radius=CORR_RADIUS):
    N, _, H, W = coords.shape
    HW = H * W
    C = f1.shape[2]
    R = 2 * radius + 1
    nplanes = len(f2_levels) * R * R
    shapes = []
    h, w = H, W
    for _ in f2_levels:
        shapes.append((h, w))
        h, w = h // 2, w // 2
    shapes = tuple(shapes)
    coords_q = coords.reshape(N, 2, HW)
    scale = 1.0 / math.sqrt(C)
    g_specs = [pl.BlockSpec((1, g.shape[1], C), lambda n: (n, 0, 0))
               for g in f2_levels]
    out = pl.pallas_call(
        functools.partial(_corr_lookup_kernel, radius, shapes, scale),
        out_shape=jax.ShapeDtypeStruct((N, nplanes, HW), jnp.bfloat16),
        grid=(N,),
        in_specs=[pl.BlockSpec((1, 2, HW), lambda n: (n, 0, 0)),
                  pl.BlockSpec((1, HW, C), lambda n: (n, 0, 0))] + g_specs,
        out_specs=pl.BlockSpec((1, nplanes, HW), lambda n: (n, 0, 0)),
        compiler_params=pltpu.CompilerParams(
            dimension_semantics=("parallel",)),
    )(coords_q, f1, *f2_levels)
    return jnp.transpose(out, (0, 2, 1)).reshape(N, H, W, nplanes)


def ctx_act(c2d):
    return _row_call(_ctx_act_kernel, [c2d], (HDIM, CDIM),
                     (jnp.bfloat16, jnp.bfloat16))


def sep_conv_gru_dir(prep, h, x, idx, ksize, padding):
    N, H, W, _ = h.shape
    M = N * H * W
    hx = jnp.concatenate([h, x], axis=-1)
    zr = conv2d(hx, prep["zr" + idx], ksize, padding=padding, act="sigmoid")
    zr2 = zr.reshape(M, 2 * HDIM)
    h2 = h.reshape(M, HDIM)
    (rh,) = _row_call(_gru_rh_kernel, [zr2, h2], (HDIM,), (jnp.bfloat16,))
    q_in = jnp.concatenate([rh.reshape(N, H, W, HDIM), x], axis=-1)
    qt = conv2d(q_in, prep["q" + idx], ksize, padding=padding, act="tanh")
    (hn,) = _row_call(_gru_blend_kernel, [zr2, qt.reshape(M, HDIM), h2],
                      (HDIM,), (jnp.bfloat16,))
    return hn.reshape(N, H, W, HDIM)


def update_block(prep, net, inp, corr, flow, info):
    fi = jnp.concatenate([jnp.transpose(flow, (0, 2, 3, 1)),
                          jnp.transpose(info, (0, 2, 3, 1))], axis=-1)
    fi_b = fi.astype(jnp.bfloat16)
    corr_b = corr.astype(jnp.bfloat16)
    cor = conv2d(corr_b, prep["convc1"], (1, 1), act="relu")
    cor = conv2d(cor, prep["convc2"], (3, 3), padding=(1, 1), act="relu")
    flo = conv2d(fi_b, prep["convf1"], (7, 7), padding=(3, 3), act="relu")
    flo = conv2d(flo, prep["convf2"], (3, 3), padding=(1, 1), act="relu")
    mot = conv2d(jnp.concatenate([cor, flo], -1), prep["conv"], (3, 3),
                 padding=(1, 1), act="relu")
    x = jnp.concatenate([inp, mot, fi_b], axis=-1)
    net = sep_conv_gru_dir(prep, net, x, "1", (1, 5), (0, 2))
    net = sep_conv_gru_dir(prep, net, x, "2", (5, 1), (2, 0))
    dm = conv2d(net, prep["dm"], (3, 3), padding=(1, 1), act="relu")
    delta = conv2d(dm[..., :256], prep["fh2"], (3, 3), padding=(1, 1),
                   act="none", out_dtype=jnp.float32)
    mask = conv2d(dm[..., 256:], prep["mh2"], (1, 1), act="none",
                  out_dtype=jnp.float32)
    return (net, jnp.transpose(mask, (0, 3, 1, 2)),
            jnp.transpose(delta, (0, 3, 1, 2)))


def unfold3x3(x):
    N, C, H, W = x.shape
    xp = jnp.pad(x, ((0, 0), (0, 0), (1, 1), (1, 1)))
    cols = [xp[:, :, ky:ky + H, kx:kx + W] for ky in range(3) for kx in range(3)]
    return jnp.stack(cols, axis=2)


def _upsample_kernel(m_ref, uf_ref, ui_ref, of_ref, oi_ref):
    m = m_ref[...]
    m = m - jnp.max(m, axis=0, keepdims=True)
    e = jnp.exp(m)
    sm = e * pl.reciprocal(jnp.sum(e, axis=0, keepdims=True), approx=True)
    uf = uf_ref[...]
    ui = ui_ref[...]
    for c in range(2):
        of_ref[c, :, :] = jnp.sum(sm * uf[:, c, :][:, None, :], axis=0)
        oi_ref[c, :, :] = jnp.sum(sm * ui[:, c, :][:, None, :], axis=0)


def upsample_flow(flow, info, mask):
    N, _, H, W = flow.shape
    P = N * H * W
    mask_k = jnp.transpose(mask.reshape(N, 9, 64, H, W),
                           (1, 2, 0, 3, 4)).reshape(9, 64, P)
    uf = unfold3x3(8.0 * flow)
    ui = unfold3x3(info)
    uf_k = jnp.transpose(uf, (2, 1, 0, 3, 4)).reshape(9, 2, P)
    ui_k = jnp.transpose(ui, (2, 1, 0, 3, 4)).reshape(9, 2, P)
    TP = 256 if P % 256 == 0 else 128
    of, oi = pl.pallas_call(
        _upsample_kernel,
        out_shape=(jax.ShapeDtypeStruct((2, 64, P), jnp.float32),
                   jax.ShapeDtypeStruct((2, 64, P), jnp.float32)),
        grid=(P // TP,),
        in_specs=[pl.BlockSpec((9, 64, TP), lambda i: (0, 0, i)),
                  pl.BlockSpec((9, 2, TP), lambda i: (0, 0, i)),
                  pl.BlockSpec((9, 2, TP), lambda i: (0, 0, i))],
        out_specs=(pl.BlockSpec((2, 64, TP), lambda i: (0, 0, i)),
                   pl.BlockSpec((2, 64, TP), lambda i: (0, 0, i))),
        compiler_params=pltpu.CompilerParams(dimension_semantics=("parallel",)),
    )(mask_k, uf_k, ui_k)

    def finish(o):
        o = o.reshape(2, 8, 8, N, H, W)
        o = jnp.transpose(o, (3, 0, 4, 1, 5, 2))
        return o.reshape(N, 2, 8 * H, 8 * W)

    return finish(of), finish(oi)


def raft_forward(prep, image1, image2, iters=2):
    N = image1.shape[0]
    x = jnp.transpose(jnp.concatenate([image1, image2], axis=0),
                      (0, 2, 3, 1)).astype(jnp.bfloat16)
    fmaps = encoder_forward(prep["fnet"], x, "instance")
    fmap1, fmap2 = fmaps[:N], fmaps[N:]
    cnet = encoder_forward(prep["cnet"],
                           jnp.transpose(image1, (0, 2, 3, 1)).astype(jnp.bfloat16),
                           "batch")
    H8, W8 = cnet.shape[1], cnet.shape[2]
    net2d, inp2d = ctx_act(cnet.reshape(N * H8 * W8, HDIM + CDIM))
    net = net2d.reshape(N, H8, W8, HDIM)
    inp = inp2d.reshape(N, H8, W8, CDIM)

    f1 = fmap1.astype(jnp.bfloat16).reshape(N, H8 * W8, -1)
    f2_levels = build_f2_pyramid(fmap2)
    coords0 = coords_grid(N, H8, W8)
    coords1 = coords0
    info = jnp.zeros_like(coords1)

    flow_predictions, info_predictions = [], []
    for _ in range(iters):
        corr = corr_lookup(f1, f2_levels, coords1, radius=CORR_RADIUS)
        flow = coords1 - coords0
        net, up_mask, delta = update_block(prep, net, inp, corr, flow, info)
        coords1 = coords1 + delta[:, :2]
        info = info + delta[:, 2:]
        flow_up, info_up = upsample_flow(coords1 - coords0, info, up_mask)
        flow_predictions.append(flow_up)
        info_predictions.append(info_up)
    return flow_predictions, info_predictions


def kernel(image1, image2,
           fnet_c1_w, fnet_c1_b, fnet_c2_w, fnet_c2_b,
           fnet_c3_w, fnet_c3_b, fnet_c4_w, fnet_c4_b,
           cnet_c1_w, cnet_c1_b, cnet_c2_w, cnet_c2_b,
           cnet_c3_w, cnet_c3_b, cnet_c4_w, cnet_c4_b,
           convc1_w, convc1_b, convc2_w, convc2_b,
           convf1_w, convf1_b, convf2_w, convf2_b,
           conv_w, conv_b, fh2_w, fh2_b,
           zr1_w, zr1_b, zr2_w, zr2_b, q1_w, q1_b, q2_w, q2_b,
           dm_w, dm_b, mh2_w, mh2_b):
    prep = {
        "fnet": {"c1": {"w": fnet_c1_w, "b": fnet_c1_b},
                 "c2": {"w": fnet_c2_w, "b": fnet_c2_b},
                 "c3": {"w": fnet_c3_w, "b": fnet_c3_b},
                 "c4": {"w": fnet_c4_w, "b": fnet_c4_b}},
        "cnet": {"c1": {"w": cnet_c1_w, "b": cnet_c1_b},
                 "c2": {"w": cnet_c2_w, "b": cnet_c2_b},
                 "c3": {"w": cnet_c3_w, "b": cnet_c3_b},
                 "c4": {"w": cnet_c4_w, "b": cnet_c4_b}},
        "convc1": {"w": convc1_w, "b": convc1_b},
        "convc2": {"w": convc2_w, "b": convc2_b},
        "convf1": {"w": convf1_w, "b": convf1_b},
        "convf2": {"w": convf2_w, "b": convf2_b},
        "conv": {"w": conv_w, "b": conv_b},
        "fh2": {"w": fh2_w, "b": fh2_b},
        "zr1": {"w": zr1_w, "b": zr1_b},
        "zr2": {"w": zr2_w, "b": zr2_b},
        "q1": {"w": q1_w, "b": q1_b},
        "q2": {"w": q2_w, "b": q2_b},
        "dm": {"w": dm_w, "b": dm_b},
        "mh2": {"w": mh2_w, "b": mh2_b},
    }
    return raft_forward(prep, image1, image2, iters=2)
